# baseline (speedup 1.0000x reference)
import functools

import jax
import jax.numpy as jnp
from jax import lax
from jax.experimental import pallas as pl
from jax.experimental.pallas import tpu as pltpu
from jax.experimental.pallas import tpu_sc as plsc

_NW = 32
_CL = 128
_BS = 128


_G = 3


def _sc_gather(n_chunks, d):
    assert n_chunks % (2 * _G) == 0
    tot = _NW * n_chunks * _CL
    rows_g = _G * _CL
    mesh = plsc.VectorSubcoreMesh(core_axis_name="c", subcore_axis_name="s")

    @functools.partial(
        pl.kernel,
        mesh=mesh,
        out_type=jax.ShapeDtypeStruct((tot, d), jnp.float32),
        scratch_types=[
            pltpu.VMEM((n_chunks + _G, _CL), jnp.int32),
            pltpu.VMEM((rows_g, d), jnp.float32),
            pltpu.VMEM((rows_g, d), jnp.float32),
            pltpu.SemaphoreType.DMA,
            pltpu.SemaphoreType.DMA,
            pltpu.SemaphoreType.DMA,
            pltpu.SemaphoreType.DMA,
        ],
    )
    def gather(table, idx, out, idx_v, buf0, buf1, g0, g1, s0, s1):
        cid = lax.axis_index("c")
        sid = lax.axis_index("s")
        wid = sid * 2 + cid
        base = wid * (n_chunks * _CL)
        pltpu.sync_copy(idx.at[wid], idx_v)

        def fire(grp, buf, sem):
            for j in range(_G):
                pltpu.make_async_copy(
                    table.at[idx_v.at[grp * _G + j]],
                    buf.at[pl.ds(j * _CL, _CL)], sem).start()

        def drain(buf, sem):
            pltpu.make_async_copy(table.at[idx_v.at[0]], buf, sem).wait()

        def store(grp, buf, sem):
            return pltpu.make_async_copy(
                buf, out.at[pl.ds(base + grp * rows_g, rows_g)], sem)

        fire(0, buf0, g0)

        def pair(p, carry):
            g = 2 * p
            drain(buf0, g0)
            store(g, buf0, s0).start()
            fire(g + 1, buf1, g1)
            store(g, buf0, s0).wait()
            drain(buf1, g1)
            store(g + 1, buf1, s1).start()
            fire(g + 2, buf0, g0)
            store(g + 1, buf1, s1).wait()
            return carry

        lax.fori_loop(0, n_chunks // (2 * _G), pair, 0)
        drain(buf0, g0)

    return gather


def _tc_body(k, eu_ref, self_ref, w1e_ref, w1s_ref, b1_ref, w2_ref, b2_ref,
             w3_ref, out_ref):
    bs = _BS
    d = self_ref.shape[1]
    e_u = eu_ref[...]
    s_f = self_ref[...]
    u1 = jnp.dot(s_f, w1s_ref[...], preferred_element_type=jnp.float32)
    u1r = jnp.broadcast_to(u1[:, None, :], (bs, k, d)).reshape(bs * k, d)
    h = jnp.dot(e_u, w1e_ref[...], preferred_element_type=jnp.float32)
    h = jnp.maximum(h + u1r + b1_ref[...], 0.0)
    h = jnp.dot(h, w2_ref[...], preferred_element_type=jnp.float32)
    h = jnp.maximum(h + b2_ref[...], 0.0)
    s = jnp.sum(h * w3_ref[...], axis=1, keepdims=True)
    s3 = s.reshape(bs, k, 1)
    m = jnp.max(s3, axis=1, keepdims=True)
    e = jnp.exp(s3 - m)
    w = e / jnp.sum(e, axis=1, keepdims=True)
    att = jnp.sum(w * e_u.reshape(bs, k, d), axis=1)
    out_ref[...] = (att + s_f) * 0.5


def _tc_mlp(bp, k, d, eu_rows, tot):
    grid = bp // _BS
    body = functools.partial(_tc_body, k)
    return pl.pallas_call(
        body,
        grid=(grid,),
        in_specs=[
            pl.BlockSpec((_BS * k, d), lambda i: (i, 0)),
            pl.BlockSpec((_BS, d), lambda i: (eu_rows // _BS + i, 0)),
            pl.BlockSpec((d, d), lambda i: (0, 0)),
            pl.BlockSpec((d, d), lambda i: (0, 0)),
            pl.BlockSpec((1, d), lambda i: (0, 0)),
            pl.BlockSpec((d, d), lambda i: (0, 0)),
            pl.BlockSpec((1, d), lambda i: (0, 0)),
            pl.BlockSpec((1, d), lambda i: (0, 0)),
        ],
        out_specs=pl.BlockSpec((_BS, d), lambda i: (i, 0)),
        out_shape=jax.ShapeDtypeStruct((bp, d), jnp.float32),
    )


def kernel(nodes, to_neighs, u2e_weight, att1_W, att1_b, att2_W, att2_b,
           att3_W, att3_b):
    b = nodes.shape[0]
    k = to_neighs.shape[1]
    d = u2e_weight.shape[1]
    bp = ((b + _BS - 1) // _BS) * _BS
    eu_rows = bp * k
    raw = eu_rows + bp
    chunk = _NW * _CL
    q = 2 * _G
    n_chunks = ((raw + chunk - 1) // chunk + q - 1) // q * q
    tot = n_chunks * chunk

    neigh = to_neighs[:, :, 0].astype(jnp.int32).reshape(b * k)
    nid = nodes.astype(jnp.int32)
    idx = jnp.concatenate([
        neigh,
        jnp.zeros((eu_rows - b * k,), jnp.int32),
        nid,
        jnp.zeros((tot - eu_rows - b,), jnp.int32),
    ]).reshape(_NW, n_chunks, _CL)
    idx = jnp.pad(idx, ((0, 0), (0, _G), (0, 0)))

    rows = _sc_gather(n_chunks, d)(u2e_weight, idx)

    w1e = att1_W[:d]
    w1s = att1_W[d:]
    out = _tc_mlp(bp, k, d, eu_rows, tot)(
        rows, rows, w1e, w1s, att1_b.reshape(1, d), att2_W,
        att2_b.reshape(1, d), att3_W.reshape(1, d))
    return out[:b]


# speedup vs baseline: 1.5419x; 1.5419x over previous
import functools

import jax
import jax.numpy as jnp
from jax import lax
from jax.experimental import pallas as pl
from jax.experimental.pallas import tpu as pltpu
from jax.experimental.pallas import tpu_sc as plsc

_NW = 32
_CL = 128
_BS = 128
_NS = 2


def _sc_gather(n_chunks, d):
    tot = _NW * n_chunks * _CL
    mesh = plsc.VectorSubcoreMesh(core_axis_name="c", subcore_axis_name="s")

    @functools.partial(
        pl.kernel,
        mesh=mesh,
        out_type=jax.ShapeDtypeStruct((tot, d), jnp.float32),
        scratch_types=[
            pltpu.VMEM((n_chunks, _CL), jnp.int32),
            pltpu.VMEM((_CL, d), jnp.float32),
            pltpu.SemaphoreType.DMA,
        ],
    )
    def gather(table, idx, out, idx_v, buf, sem):
        cid = lax.axis_index("c")
        sid = lax.axis_index("s")
        wid = sid * 2 + cid
        base = wid * (n_chunks * _CL)
        pltpu.sync_copy(idx.at[wid], idx_v)

        def body(c, carry):
            pltpu.async_copy(table.at[idx_v.at[c]], buf, sem).wait()
            pltpu.sync_copy(buf, out.at[pl.ds(base + c * _CL, _CL)])
            return carry

        lax.fori_loop(0, n_chunks, body, 0)

    return gather


def _tc_body(k, eu_ref, self_ref, w1e_ref, w1s_ref, b1_ref, w2_ref, b2_ref,
             w3_ref, out_ref):
    bs = _BS
    d = self_ref.shape[1]
    e_u = eu_ref[...]
    s_f = self_ref[...]
    u1 = jnp.dot(s_f, w1s_ref[...], preferred_element_type=jnp.float32)
    u1r = jnp.broadcast_to(u1[:, None, :], (bs, k, d)).reshape(bs * k, d)
    h = jnp.dot(e_u, w1e_ref[...], preferred_element_type=jnp.float32)
    h = jnp.maximum(h + u1r + b1_ref[...], 0.0)
    h = jnp.dot(h, w2_ref[...], preferred_element_type=jnp.float32)
    h = jnp.maximum(h + b2_ref[...], 0.0)
    s = jnp.sum(h * w3_ref[...], axis=1, keepdims=True)
    s3 = s.reshape(bs, k, 1)
    m = jnp.max(s3, axis=1, keepdims=True)
    e = jnp.exp(s3 - m)
    w = e / jnp.sum(e, axis=1, keepdims=True)
    att = jnp.sum(w * e_u.reshape(bs, k, d), axis=1)
    out_ref[...] = (att + s_f) * 0.5


def _tc_mlp(bp, k, d, eu_rows):
    grid = bp // _BS
    body = functools.partial(_tc_body, k)
    return pl.pallas_call(
        body,
        grid=(grid,),
        in_specs=[
            pl.BlockSpec((_BS * k, d), lambda i: (i, 0)),
            pl.BlockSpec((_BS, d), lambda i: (eu_rows // _BS + i, 0)),
            pl.BlockSpec((d, d), lambda i: (0, 0)),
            pl.BlockSpec((d, d), lambda i: (0, 0)),
            pl.BlockSpec((1, d), lambda i: (0, 0)),
            pl.BlockSpec((d, d), lambda i: (0, 0)),
            pl.BlockSpec((1, d), lambda i: (0, 0)),
            pl.BlockSpec((1, d), lambda i: (0, 0)),
        ],
        out_specs=pl.BlockSpec((_BS, d), lambda i: (i, 0)),
        out_shape=jax.ShapeDtypeStruct((bp, d), jnp.float32),
    )


def kernel(nodes, to_neighs, u2e_weight, att1_W, att1_b, att2_W, att2_b,
           att3_W, att3_b):
    b = nodes.shape[0]
    k = to_neighs.shape[1]
    d = u2e_weight.shape[1]
    bs = (b + _NS - 1) // _NS
    bp = ((bs + _BS - 1) // _BS) * _BS
    eu_rows = bp * k
    raw = eu_rows + bp
    chunk = _NW * _CL
    n_chunks = (raw + chunk - 1) // chunk
    tot = n_chunks * chunk

    neigh = to_neighs[:, :, 0].astype(jnp.int32).reshape(b, k)
    nid = nodes.astype(jnp.int32)
    w1e = att1_W[:d]
    w1s = att1_W[d:]
    b1 = att1_b.reshape(1, d)
    b2 = att2_b.reshape(1, d)
    w3 = att3_W.reshape(1, d)

    gather_fn = _sc_gather(n_chunks, d)
    mlp_fn = _tc_mlp(bp, k, d, eu_rows)

    outs = []
    for j in range(_NS):
        lo = j * bs
        ns = min(bs, b - lo)
        idx = jnp.concatenate([
            neigh[lo:lo + ns].reshape(ns * k),
            jnp.zeros((eu_rows - ns * k,), jnp.int32),
            nid[lo:lo + ns],
            jnp.zeros((tot - eu_rows - ns,), jnp.int32),
        ]).reshape(_NW, n_chunks, _CL)
        rows = gather_fn(u2e_weight, idx)
        out = mlp_fn(rows, rows, w1e, w1s, b1, att2_W, b2, w3)
        outs.append(out[:ns])
    return jnp.concatenate(outs, axis=0)


# speedup vs baseline: 1.9107x; 1.2392x over previous
import functools

import jax
import jax.numpy as jnp
from jax import lax
from jax.experimental import pallas as pl
from jax.experimental.pallas import tpu as pltpu
from jax.experimental.pallas import tpu_sc as plsc

_NW = 32
_CL = 128
_BS = 128
def _sc_gather(n_chunks, d):
    assert n_chunks % 2 == 0
    tot = _NW * n_chunks * _CL
    mesh = plsc.VectorSubcoreMesh(core_axis_name="c", subcore_axis_name="s")

    @functools.partial(
        pl.kernel,
        mesh=mesh,
        out_type=jax.ShapeDtypeStruct((tot, d), jnp.float32),
        scratch_types=[
            pltpu.VMEM((n_chunks + 1, _CL), jnp.int32),
            pltpu.VMEM((_CL, d), jnp.float32),
            pltpu.VMEM((_CL, d), jnp.float32),
            pltpu.SemaphoreType.DMA,
            pltpu.SemaphoreType.DMA,
        ],
    )
    def gather(table, idx, out, idx_v, buf0, buf1, g0, g1):
        cid = lax.axis_index("c")
        sid = lax.axis_index("s")
        wid = sid * 2 + cid
        base = wid * (n_chunks * _CL)
        pltpu.sync_copy(idx.at[wid], idx_v)

        pltpu.async_copy(table.at[idx_v.at[0]], buf0, g0)

        def body(p, carry):
            c = 2 * p
            pltpu.async_copy(table.at[idx_v.at[c + 1]], buf1, g1)
            pltpu.make_async_copy(table.at[idx_v.at[c]], buf0, g0).wait()
            pltpu.sync_copy(buf0, out.at[pl.ds(base + c * _CL, _CL)])
            pltpu.async_copy(table.at[idx_v.at[c + 2]], buf0, g0)
            pltpu.make_async_copy(table.at[idx_v.at[c + 1]], buf1, g1).wait()
            pltpu.sync_copy(buf1, out.at[pl.ds(base + (c + 1) * _CL, _CL)])
            return carry

        lax.fori_loop(0, n_chunks // 2, body, 0)
        pltpu.make_async_copy(table.at[idx_v.at[n_chunks]], buf0, g0).wait()

    return gather


def _tc_body(k, eu_ref, self_ref, w1e_ref, w1s_ref, b1_ref, w2_ref, b2_ref,
             w3_ref, out_ref):
    bs = _BS
    d = self_ref.shape[1]
    e_u = eu_ref[...]
    s_f = self_ref[...]
    u1 = jnp.dot(s_f, w1s_ref[...], preferred_element_type=jnp.float32)
    u1r = jnp.broadcast_to(u1[:, None, :], (bs, k, d)).reshape(bs * k, d)
    h = jnp.dot(e_u, w1e_ref[...], preferred_element_type=jnp.float32)
    h = jnp.maximum(h + u1r + b1_ref[...], 0.0)
    h = jnp.dot(h, w2_ref[...], preferred_element_type=jnp.float32)
    h = jnp.maximum(h + b2_ref[...], 0.0)
    s = jnp.sum(h * w3_ref[...], axis=1, keepdims=True)
    s3 = s.reshape(bs, k, 1)
    m = jnp.max(s3, axis=1, keepdims=True)
    e = jnp.exp(s3 - m)
    w = e / jnp.sum(e, axis=1, keepdims=True)
    att = jnp.sum(w * e_u.reshape(bs, k, d), axis=1)
    out_ref[...] = (att + s_f) * 0.5


def _tc_mlp(bp, k, d, eu_rows):
    grid = bp // _BS
    body = functools.partial(_tc_body, k)
    return pl.pallas_call(
        body,
        grid=(grid,),
        in_specs=[
            pl.BlockSpec((_BS * k, d), lambda i: (i, 0)),
            pl.BlockSpec((_BS, d), lambda i: (eu_rows // _BS + i, 0)),
            pl.BlockSpec((d, d), lambda i: (0, 0)),
            pl.BlockSpec((d, d), lambda i: (0, 0)),
            pl.BlockSpec((1, d), lambda i: (0, 0)),
            pl.BlockSpec((d, d), lambda i: (0, 0)),
            pl.BlockSpec((1, d), lambda i: (0, 0)),
            pl.BlockSpec((1, d), lambda i: (0, 0)),
        ],
        out_specs=pl.BlockSpec((_BS, d), lambda i: (i, 0)),
        out_shape=jax.ShapeDtypeStruct((bp, d), jnp.float32),
    )


def kernel(nodes, to_neighs, u2e_weight, att1_W, att1_b, att2_W, att2_b,
           att3_W, att3_b):
    b = nodes.shape[0]
    k = to_neighs.shape[1]
    d = u2e_weight.shape[1]
    bp = ((b + _BS - 1) // _BS) * _BS
    eu_rows = bp * k
    raw = eu_rows + bp
    chunk = _NW * _CL
    n_chunks = (raw + chunk - 1) // chunk
    n_chunks += n_chunks % 2
    tot = n_chunks * chunk

    neigh = to_neighs[:, :, 0].astype(jnp.int32).reshape(b * k)
    nid = nodes.astype(jnp.int32)
    idx = jnp.concatenate([
        neigh,
        jnp.zeros((eu_rows - b * k,), jnp.int32),
        nid,
        jnp.zeros((tot - eu_rows - b,), jnp.int32),
    ]).reshape(_NW, n_chunks, _CL)
    idx = jnp.pad(idx, ((0, 0), (0, 1), (0, 0)))

    rows = _sc_gather(n_chunks, d)(u2e_weight, idx)

    w1e = att1_W[:d]
    w1s = att1_W[d:]
    out = _tc_mlp(bp, k, d, eu_rows)(
        rows, rows, w1e, w1s, att1_b.reshape(1, d), att2_W,
        att2_b.reshape(1, d), att3_W.reshape(1, d))
    return out[:b]


# speedup vs baseline: 2.3063x; 1.2070x over previous
import functools

import jax
import jax.numpy as jnp
from jax import lax
from jax.experimental import pallas as pl
from jax.experimental.pallas import tpu as pltpu
from jax.experimental.pallas import tpu_sc as plsc

_NW = 32
_CL = 128
_BS = 128
def _sc_gather(n_chunks, d):
    assert n_chunks % 2 == 0
    tot = _NW * n_chunks * _CL
    mesh = plsc.VectorSubcoreMesh(core_axis_name="c", subcore_axis_name="s")

    @functools.partial(
        pl.kernel,
        mesh=mesh,
        out_type=jax.ShapeDtypeStruct((tot, d), jnp.float32),
        scratch_types=[
            pltpu.VMEM((n_chunks + 1, _CL), jnp.int32),
            pltpu.VMEM((_CL, d), jnp.float32),
            pltpu.VMEM((_CL, d), jnp.float32),
            pltpu.SemaphoreType.DMA,
            pltpu.SemaphoreType.DMA,
        ],
    )
    def gather(table, idx, out, idx_v, buf0, buf1, g0, g1):
        cid = lax.axis_index("c")
        sid = lax.axis_index("s")
        wid = sid * 2 + cid
        base = wid * (n_chunks * _CL)
        pltpu.sync_copy(idx.at[wid], idx_v)

        def chunk_in(c, buf, sem):
            pltpu.async_copy(table.at[idx_v.at[c]], buf, sem).wait()

        def store(c, buf, sem):
            return pltpu.make_async_copy(
                buf, out.at[pl.ds(base + c * _CL, _CL)], sem)

        chunk_in(0, buf0, g0)
        store(0, buf0, g0).start()
        chunk_in(1, buf1, g1)
        store(1, buf1, g1).start()

        def body(q, carry):
            c = 2 * q
            store(c - 2, buf0, g0).wait()
            chunk_in(c, buf0, g0)
            store(c, buf0, g0).start()
            store(c - 1, buf1, g1).wait()
            chunk_in(c + 1, buf1, g1)
            store(c + 1, buf1, g1).start()
            return carry

        lax.fori_loop(1, n_chunks // 2, body, 0)
        store(n_chunks - 2, buf0, g0).wait()
        store(n_chunks - 1, buf1, g1).wait()

    return gather


def _tc_body(k, eu_ref, self_ref, w1e_ref, w1s_ref, b1_ref, w2_ref, b2_ref,
             w3_ref, out_ref):
    bs = _BS
    d = self_ref.shape[1]
    e_u = eu_ref[...]
    s_f = self_ref[...]
    u1 = jnp.dot(s_f, w1s_ref[...], preferred_element_type=jnp.float32)
    u1r = jnp.broadcast_to(u1[:, None, :], (bs, k, d)).reshape(bs * k, d)
    h = jnp.dot(e_u, w1e_ref[...], preferred_element_type=jnp.float32)
    h = jnp.maximum(h + u1r + b1_ref[...], 0.0)
    h = jnp.dot(h, w2_ref[...], preferred_element_type=jnp.float32)
    h = jnp.maximum(h + b2_ref[...], 0.0)
    s = jnp.sum(h * w3_ref[...], axis=1, keepdims=True)
    s3 = s.reshape(bs, k, 1)
    m = jnp.max(s3, axis=1, keepdims=True)
    e = jnp.exp(s3 - m)
    w = e / jnp.sum(e, axis=1, keepdims=True)
    att = jnp.sum(w * e_u.reshape(bs, k, d), axis=1)
    out_ref[...] = (att + s_f) * 0.5


def _tc_mlp(bp, k, d, eu_rows):
    grid = bp // _BS
    body = functools.partial(_tc_body, k)
    return pl.pallas_call(
        body,
        grid=(grid,),
        in_specs=[
            pl.BlockSpec((_BS * k, d), lambda i: (i, 0)),
            pl.BlockSpec((_BS, d), lambda i: (eu_rows // _BS + i, 0)),
            pl.BlockSpec((d, d), lambda i: (0, 0)),
            pl.BlockSpec((d, d), lambda i: (0, 0)),
            pl.BlockSpec((1, d), lambda i: (0, 0)),
            pl.BlockSpec((d, d), lambda i: (0, 0)),
            pl.BlockSpec((1, d), lambda i: (0, 0)),
            pl.BlockSpec((1, d), lambda i: (0, 0)),
        ],
        out_specs=pl.BlockSpec((_BS, d), lambda i: (i, 0)),
        out_shape=jax.ShapeDtypeStruct((bp, d), jnp.float32),
    )


def kernel(nodes, to_neighs, u2e_weight, att1_W, att1_b, att2_W, att2_b,
           att3_W, att3_b):
    b = nodes.shape[0]
    k = to_neighs.shape[1]
    d = u2e_weight.shape[1]
    bp = ((b + _BS - 1) // _BS) * _BS
    eu_rows = bp * k
    raw = eu_rows + bp
    chunk = _NW * _CL
    n_chunks = (raw + chunk - 1) // chunk
    n_chunks += n_chunks % 2
    tot = n_chunks * chunk

    neigh = to_neighs[:, :, 0].astype(jnp.int32).reshape(b * k)
    nid = nodes.astype(jnp.int32)
    idx = jnp.concatenate([
        neigh,
        jnp.zeros((eu_rows - b * k,), jnp.int32),
        nid,
        jnp.zeros((tot - eu_rows - b,), jnp.int32),
    ]).reshape(_NW, n_chunks, _CL)
    idx = jnp.pad(idx, ((0, 0), (0, 1), (0, 0)))

    rows = _sc_gather(n_chunks, d)(u2e_weight, idx)

    w1e = att1_W[:d]
    w1s = att1_W[d:]
    out = _tc_mlp(bp, k, d, eu_rows)(
        rows, rows, w1e, w1s, att1_b.reshape(1, d), att2_W,
        att2_b.reshape(1, d), att3_W.reshape(1, d))
    return out[:b]
